# baseline (speedup 1.0000x reference)
import functools

import jax
import jax.numpy as jnp
from jax import lax
from jax.experimental import pallas as pl
from jax.experimental.pallas import tpu as pltpu
from jax.experimental.pallas import tpu_sc as plsc

VOCAB = 1000000
D = 32
B = 16384

_info = plsc.get_sparse_core_info()
NC = _info.num_cores
NS = _info.num_subcores
NW = NC * NS
BPW = B // NW
CH = 128
NCH = BPW // CH

_mesh = plsc.VectorSubcoreMesh(core_axis_name="c", subcore_axis_name="s")


@functools.partial(
    pl.kernel,
    mesh=_mesh,
    compiler_params=pltpu.CompilerParams(use_tc_tiling_on_sc=False),
    out_type=jax.ShapeDtypeStruct((B, 2, D), jnp.float32),
    scratch_types=[
        pltpu.VMEM((NCH, CH), jnp.int32),
        pltpu.VMEM((NCH, CH), jnp.int32),
        pltpu.VMEM((BPW, D), jnp.float32),
        pltpu.VMEM((BPW, D), jnp.float32),
        pltpu.SemaphoreType.DMA,
    ],
)
def _lookup_concat(uid_hbm, sex_hbm, utab_hbm, stab_hbm, out_hbm,
                   uidx_v, sidx_v, urows_v, srows_v, sem):
    wid = lax.axis_index("s") * NC + lax.axis_index("c")
    base = wid * BPW
    pltpu.sync_copy(uid_hbm.at[wid], uidx_v)
    pltpu.sync_copy(sex_hbm.at[wid], sidx_v)
    copies = []
    for j in range(NCH):
        copies.append(pltpu.async_copy(
            utab_hbm.at[uidx_v.at[j]], urows_v.at[pl.ds(j * CH, CH)], sem))
        copies.append(pltpu.async_copy(
            stab_hbm.at[sidx_v.at[j]], srows_v.at[pl.ds(j * CH, CH)], sem))
    for c in copies:
        c.wait()
    pltpu.sync_copy(urows_v, out_hbm.at[pl.ds(base, BPW), 0])
    pltpu.sync_copy(srows_v, out_hbm.at[pl.ds(base, BPW), 1])


def kernel(user_id, sex, user_table, sex_table):
    uid = user_id.astype(jnp.int32).reshape(NW, NCH, CH)
    sx = sex.astype(jnp.int32).reshape(NW, NCH, CH)
    out = _lookup_concat(uid, sx, user_table, sex_table)
    return out.reshape(B, 2 * D)


# speedup vs baseline: 1.2773x; 1.2773x over previous
import functools

import jax
import jax.numpy as jnp
from jax import lax
from jax.experimental import pallas as pl
from jax.experimental.pallas import tpu as pltpu
from jax.experimental.pallas import tpu_sc as plsc

VOCAB = 1000000
D = 32
B = 16384

_info = plsc.get_sparse_core_info()
NC = _info.num_cores
NS = _info.num_subcores
NW = NC * NS
BPW = B // NW
CH = 128
NCH = BPW // CH

_mesh = plsc.VectorSubcoreMesh(core_axis_name="c", subcore_axis_name="s")


@functools.partial(
    pl.kernel,
    mesh=_mesh,
    compiler_params=pltpu.CompilerParams(
        use_tc_tiling_on_sc=False, needs_layout_passes=False),
    out_type=jax.ShapeDtypeStruct((B, 2, D), jnp.float32),
    scratch_types=[
        pltpu.VMEM((NCH, CH), jnp.int32),
        pltpu.VMEM((BPW,), jnp.int32),
        pltpu.VMEM((BPW, D), jnp.float32),
        pltpu.VMEM((BPW, D), jnp.float32),
        pltpu.VMEM((64,), jnp.float32),
        pltpu.SemaphoreType.DMA,
    ],
)
def _lookup_concat(uid_hbm, sex_hbm, utab_hbm, stab_hbm, out_hbm,
                   uidx_v, sidx_v, urows_v, srows_v, stab_v, sem):
    wid = lax.axis_index("s") * NC + lax.axis_index("c")
    base = wid * BPW
    pltpu.sync_copy(uid_hbm.at[wid], uidx_v)
    pltpu.sync_copy(sex_hbm.at[wid], sidx_v)
    pltpu.sync_copy(stab_hbm, stab_v)
    copies = []
    for j in range(NCH):
        copies.append(pltpu.async_copy(
            utab_hbm.at[uidx_v.at[j]], urows_v.at[pl.ds(j * CH, CH)], sem))

    lanes = lax.iota(jnp.int32, 16)
    s0a = stab_v[pl.ds(0, 16)]
    s0b = stab_v[pl.ds(16, 16)]
    s1a = stab_v[pl.ds(32, 16)]
    s1b = stab_v[pl.ds(48, 16)]

    def body(i, _):
        s = plsc.load_gather(sidx_v, [lanes * 0 + i]) == 1
        srows_v[i, pl.ds(0, 16)] = jnp.where(s, s1a, s0a)
        srows_v[i, pl.ds(16, 16)] = jnp.where(s, s1b, s0b)
        return 0

    lax.fori_loop(0, BPW, body, 0)

    for c in copies:
        c.wait()
    pltpu.sync_copy(urows_v, out_hbm.at[pl.ds(base, BPW), 0])
    pltpu.sync_copy(srows_v, out_hbm.at[pl.ds(base, BPW), 1])


def kernel(user_id, sex, user_table, sex_table):
    uid = user_id.astype(jnp.int32).reshape(NW, NCH, CH)
    sx = sex.astype(jnp.int32).reshape(NW, BPW)
    stab = sex_table.reshape(64)
    out = _lookup_concat(uid, sx, user_table, stab)
    return out.reshape(B, 2 * D)


# speedup vs baseline: 1.3547x; 1.0606x over previous
import functools

import jax
import jax.numpy as jnp
from jax import lax
from jax.experimental import pallas as pl
from jax.experimental.pallas import tpu as pltpu
from jax.experimental.pallas import tpu_sc as plsc

VOCAB = 1000000
D = 32
B = 16384
VG = VOCAB // 4

_info = plsc.get_sparse_core_info()
NC = _info.num_cores
NS = _info.num_subcores
NW = NC * NS
BPW = B // NW
CH = 128
NCH = BPW // CH

_mesh = plsc.VectorSubcoreMesh(core_axis_name="c", subcore_axis_name="s")


@functools.partial(
    pl.kernel,
    mesh=_mesh,
    compiler_params=pltpu.CompilerParams(needs_layout_passes=False),
    out_type=jax.ShapeDtypeStruct((2 * D, B), jnp.float32),
    scratch_types=[
        pltpu.VMEM((BPW,), jnp.int32),
        pltpu.VMEM((BPW,), jnp.int32),
        pltpu.VMEM((BPW,), jnp.int32),
        pltpu.VMEM((BPW, 128), jnp.float32),
        pltpu.VMEM((64,), jnp.float32),
        pltpu.VMEM((2 * D, BPW), jnp.float32),
        pltpu.SemaphoreType.DMA,
    ],
)
def _lookup_concat(uid_hbm, sex_hbm, utab_hbm, stab_hbm, out_hbm,
                   uid_v, gid_v, sex_v, grp_v, stab_v, out_v, sem):
    wid = lax.axis_index("s") * NC + lax.axis_index("c")
    base = wid * BPW
    pltpu.sync_copy(uid_hbm.at[wid], uid_v)
    pltpu.sync_copy(sex_hbm.at[wid], sex_v)
    pltpu.sync_copy(stab_hbm, stab_v)

    for k in range(BPW // 16):
        gid_v[pl.ds(k * 16, 16)] = lax.shift_right_logical(
            uid_v[pl.ds(k * 16, 16)], 2)

    copies = []
    for j in range(NCH):
        copies.append(pltpu.async_copy(
            utab_hbm.at[gid_v.at[pl.ds(j * CH, CH)]],
            grp_v.at[pl.ds(j * CH, CH)], sem))
    for c in copies:
        c.wait()

    lanes = lax.iota(jnp.int32, 16)

    def body(g, _):
        i0 = g * 16
        u16 = uid_v[pl.ds(i0, 16)]
        s16 = sex_v[pl.ds(i0, 16)]
        rows = lanes + i0
        ucol = (u16 & 3) * D
        scol = s16 * D
        for c in range(D):
            x = plsc.load_gather(grp_v, [rows, ucol + c])
            out_v[c, pl.ds(i0, 16)] = x
            y = plsc.load_gather(stab_v, [scol + c])
            out_v[D + c, pl.ds(i0, 16)] = y
        return 0

    lax.fori_loop(0, BPW // 16, body, 0)
    pltpu.sync_copy(out_v, out_hbm.at[:, pl.ds(base, BPW)])


def kernel(user_id, sex, user_table, sex_table):
    uid = user_id.astype(jnp.int32).reshape(NW, BPW)
    sx = sex.astype(jnp.int32).reshape(NW, BPW)
    utab = user_table.reshape(VG, 128)
    stab = sex_table.reshape(64)
    out_t = _lookup_concat(uid, sx, utab, stab)
    return out_t.T


# speedup vs baseline: 2.1117x; 1.5588x over previous
import functools

import jax
import jax.numpy as jnp
from jax import lax
from jax.experimental import pallas as pl
from jax.experimental.pallas import tpu as pltpu
from jax.experimental.pallas import tpu_sc as plsc

VOCAB = 1000000
D = 32
B = 16384

_info = plsc.get_sparse_core_info()
NC = _info.num_cores
NS = _info.num_subcores
NW = NC * NS
BPW = B // NW
CH = 32
NCH = BPW // CH
GPC = CH // 16

_mesh = plsc.VectorSubcoreMesh(core_axis_name="c", subcore_axis_name="s")


@functools.partial(
    pl.kernel,
    mesh=_mesh,
    compiler_params=pltpu.CompilerParams(needs_layout_passes=False),
    out_type=jax.ShapeDtypeStruct((2 * D, B), jnp.float32),
    scratch_types=[
        pltpu.VMEM((BPW,), jnp.int32),
        pltpu.VMEM((BPW,), jnp.int32),
        pltpu.VMEM((64,), jnp.float32),
        pltpu.VMEM((CH, 8, D), jnp.float32),
        pltpu.VMEM((CH, 8, D), jnp.float32),
        pltpu.VMEM((CH * 8, D), jnp.float32),
        pltpu.VMEM((2 * D, BPW), jnp.float32),
        pltpu.SemaphoreType.DMA,
        pltpu.SemaphoreType.DMA,
    ],
)
def _lookup_concat(uid_hbm, sex_hbm, utab_hbm, stab_hbm, out_hbm,
                   uid_v, sex_v, stab_v, grp0_v, grp1_v, dummy_v, out_v,
                   sem0, sem1):
    wid = lax.axis_index("s") * NC + lax.axis_index("c")
    base = wid * BPW
    pltpu.sync_copy(uid_hbm.at[wid], uid_v)
    pltpu.sync_copy(sex_hbm.at[wid], sex_v)
    pltpu.sync_copy(stab_hbm, stab_v)

    lanes = lax.iota(jnp.int32, 16)
    bufs = (grp0_v, grp1_v)
    sems = (sem0, sem1)

    def fire_chunk(q, buf, sem):
        def fg(g, _):
            u16 = uid_v[pl.ds(q * CH + g * 16, 16)]
            for j in range(16):
                off = pl.multiple_of((u16[j] >> 3) * 8, 8)
                pltpu.async_copy(
                    utab_hbm.at[pl.ds(off, 8), pl.ds(0, D)],
                    buf.at[g * 16 + j], sem)
            return 0
        lax.fori_loop(0, GPC, fg, 0)

    def drain(sem):
        pltpu.make_async_copy(
            utab_hbm.at[pl.ds(0, CH * 8), pl.ds(0, D)], dummy_v, sem).wait()

    def extract_chunk(q, buf):
        def eg(g, _):
            i0 = q * CH + g * 16
            u16 = uid_v[pl.ds(i0, 16)]
            s16 = sex_v[pl.ds(i0, 16)]
            pos = lanes + g * 16
            sub = u16 & 7
            scol = s16 * D
            for c in range(D):
                x = plsc.load_gather(buf, [pos, sub, lanes * 0 + c])
                out_v[c, pl.ds(i0, 16)] = x
                y = plsc.load_gather(stab_v, [scol + c])
                out_v[D + c, pl.ds(i0, 16)] = y
            return 0
        lax.fori_loop(0, GPC, eg, 0)

    fire_chunk(0, bufs[0], sems[0])
    for q in range(1, NCH):
        fire_chunk(q, bufs[q % 2], sems[q % 2])
        drain(sems[(q - 1) % 2])
        extract_chunk(q - 1, bufs[(q - 1) % 2])
    drain(sems[(NCH - 1) % 2])
    extract_chunk(NCH - 1, bufs[(NCH - 1) % 2])

    pltpu.sync_copy(out_v, out_hbm.at[:, pl.ds(base, BPW)])


def kernel(user_id, sex, user_table, sex_table):
    uid = user_id.astype(jnp.int32).reshape(NW, BPW)
    sx = sex.astype(jnp.int32).reshape(NW, BPW)
    stab = sex_table.reshape(64)
    out_t = _lookup_concat(uid, sx, user_table, stab)
    return out_t.T


# speedup vs baseline: 5.0573x; 2.3949x over previous
import functools

import jax
import jax.numpy as jnp
from jax import lax
from jax.experimental import pallas as pl
from jax.experimental.pallas import tpu as pltpu
from jax.experimental.pallas import tpu_sc as plsc

VOCAB = 1000000
D = 32
B = 16384

_info = plsc.get_sparse_core_info()
NC = _info.num_cores
NS = _info.num_subcores
NW = NC * NS
BPW = B // NW
CH = 8
NCH = BPW // CH

_mesh = plsc.VectorSubcoreMesh(core_axis_name="c", subcore_axis_name="s")


@functools.partial(
    pl.kernel,
    mesh=_mesh,
    compiler_params=pltpu.CompilerParams(needs_layout_passes=False),
    out_type=jax.ShapeDtypeStruct((2 * D, B), jnp.float32),
    scratch_types=[
        pltpu.VMEM((BPW,), jnp.int32),
        pltpu.VMEM((BPW,), jnp.int32),
        pltpu.VMEM((64,), jnp.float32),
        pltpu.VMEM((CH, D, 128), jnp.float32),
        pltpu.VMEM((CH, D, 128), jnp.float32),
        pltpu.VMEM((D, 128), jnp.float32),
        pltpu.VMEM((2 * D, BPW), jnp.float32),
        pltpu.SemaphoreType.DMA,
        pltpu.SemaphoreType.DMA,
    ],
)
def _lookup_concat(uid_hbm, sex_hbm, utab_hbm, stab_hbm, out_hbm,
                   uid_v, sex_v, stab_v, buf0_v, buf1_v, dummy_v, out_v,
                   sem0, sem1):
    wid = lax.axis_index("s") * NC + lax.axis_index("c")
    base = wid * BPW
    pltpu.sync_copy(uid_hbm.at[wid], uid_v)
    pltpu.sync_copy(sex_hbm.at[wid], sex_v)
    pltpu.sync_copy(stab_hbm, stab_v)

    lanes = lax.iota(jnp.int32, 16)

    def fire_chunk(q, j0, buf, sem):
        u16 = uid_v[pl.ds((q >> 1) * 16, 16)]
        for j in range(CH):
            u = u16[j0 + j]
            col = pl.multiple_of((u >> 7) * 128, 128)
            pltpu.async_copy(
                utab_hbm.at[pl.ds(0, D), pl.ds(col, 128)], buf.at[j], sem)

    def drain(sem):
        for _ in range(CH):
            pltpu.make_async_copy(
                utab_hbm.at[pl.ds(0, D), pl.ds(0, 128)], dummy_v, sem).wait()

    def extract_chunk(q, j0, buf):
        u16 = uid_v[pl.ds((q >> 1) * 16, 16)]
        s16 = sex_v[pl.ds((q >> 1) * 16, 16)]
        for j in range(CH):
            u = u16[j0 + j]
            s = s16[j0 + j]
            i = q * CH + j
            lane = u & 127
            col16 = lanes * 0 + i
            for h in range(D // 16):
                x = plsc.load_gather(
                    buf, [lanes * 0 + j, lanes + h * 16, lanes * 0 + lane])
                plsc.store_scatter(out_v, [lanes + h * 16, col16], x)
                y = plsc.load_gather(stab_v, [s * D + h * 16 + lanes])
                plsc.store_scatter(out_v, [lanes + D + h * 16, col16], y)

    fire_chunk(0, 0, buf0_v, sem0)

    def body(k, _):
        @pl.when(k % 2 == 0)
        def _even():
            @pl.when(k < NCH - 1)
            def _f():
                fire_chunk(k + 1, CH, buf1_v, sem1)
            drain(sem0)
            extract_chunk(k, 0, buf0_v)

        @pl.when(k % 2 == 1)
        def _odd():
            @pl.when(k < NCH - 1)
            def _f():
                fire_chunk(k + 1, 0, buf0_v, sem0)
            drain(sem1)
            extract_chunk(k, CH, buf1_v)
        return 0

    lax.fori_loop(0, NCH, body, 0)
    pltpu.sync_copy(out_v, out_hbm.at[:, pl.ds(base, BPW)])


def kernel(user_id, sex, user_table, sex_table):
    uid = user_id.astype(jnp.int32).reshape(NW, BPW)
    sx = sex.astype(jnp.int32).reshape(NW, BPW)
    utab_t = user_table.T
    stab = sex_table.reshape(64)
    out_t = _lookup_concat(uid, sx, utab_t, stab)
    return out_t.T


# speedup vs baseline: 5.4981x; 1.0872x over previous
import functools

import jax
import jax.numpy as jnp
from jax import lax
from jax.experimental import pallas as pl
from jax.experimental.pallas import tpu as pltpu
from jax.experimental.pallas import tpu_sc as plsc

VOCAB = 1000000
D = 32
B = 16384

_info = plsc.get_sparse_core_info()
NC = _info.num_cores
NS = _info.num_subcores
NW = NC * NS
BPW = B // NW
CH = 8
NCH = BPW // CH
OUTW = 256

_mesh = plsc.VectorSubcoreMesh(core_axis_name="c", subcore_axis_name="s")


@functools.partial(
    pl.kernel,
    mesh=_mesh,
    compiler_params=pltpu.CompilerParams(needs_layout_passes=False),
    out_type=jax.ShapeDtypeStruct((2 * D, B), jnp.float32),
    scratch_types=[
        pltpu.VMEM((BPW,), jnp.int32),
        pltpu.VMEM((BPW,), jnp.int32),
        pltpu.VMEM((64,), jnp.float32),
        pltpu.VMEM((CH, D, 128), jnp.float32),
        pltpu.VMEM((CH, D, 128), jnp.float32),
        pltpu.VMEM((CH, D, 128), jnp.float32),
        pltpu.VMEM((D, 128), jnp.float32),
        pltpu.VMEM((2 * D, OUTW), jnp.float32),
        pltpu.SemaphoreType.DMA,
        pltpu.SemaphoreType.DMA,
        pltpu.SemaphoreType.DMA,
    ],
)
def _lookup_concat(uid_hbm, sex_hbm, utab_hbm, stab_hbm, out_hbm,
                   uid_v, sex_v, stab_v, buf0_v, buf1_v, buf2_v, dummy_v,
                   out_v, sem0, sem1, sem2):
    wid = lax.axis_index("s") * NC + lax.axis_index("c")
    base = wid * BPW
    pltpu.sync_copy(uid_hbm.at[wid], uid_v)
    pltpu.sync_copy(sex_hbm.at[wid], sex_v)
    pltpu.sync_copy(stab_hbm, stab_v)

    lanes = lax.iota(jnp.int32, 16)
    bufs = (buf0_v, buf1_v, buf2_v)
    sems = (sem0, sem1, sem2)

    def fire_chunk(q, j0, buf, sem):
        u16 = uid_v[pl.ds((q >> 1) * 16, 16)]
        for j in range(CH):
            u = u16[j0 + j]
            col = pl.multiple_of((u >> 7) * 128, 128)
            pltpu.async_copy(
                utab_hbm.at[pl.ds(0, D), pl.ds(col, 128)], buf.at[j], sem)

    def drain(sem):
        for _ in range(CH):
            pltpu.make_async_copy(
                utab_hbm.at[pl.ds(0, D), pl.ds(0, 128)], dummy_v, sem).wait()

    def extract_chunk(q, j0, buf):
        u16 = uid_v[pl.ds((q >> 1) * 16, 16)]
        s16 = sex_v[pl.ds((q >> 1) * 16, 16)]
        for j in range(CH):
            u = u16[j0 + j]
            s = s16[j0 + j]
            lane = u & 127
            col16 = lanes * 0 + ((q * CH + j) & (OUTW - 1))
            for h in range(D // 16):
                x = plsc.load_gather(
                    buf, [lanes * 0 + j, lanes + h * 16, lanes * 0 + lane])
                plsc.store_scatter(out_v, [lanes + h * 16, col16], x)
                y = plsc.load_gather(stab_v, [s * D + h * 16 + lanes])
                plsc.store_scatter(out_v, [lanes + D + h * 16, col16], y)

    fire_chunk(0, 0, bufs[0], sems[0])
    fire_chunk(1, CH, bufs[1], sems[1])

    def body(k, _):
        for r in range(6):
            @pl.when(k % 6 == r)
            def _step(r=r):
                @pl.when(k < NCH - 2)
                def _f():
                    fire_chunk(k + 2, ((r + 2) % 2) * CH,
                               bufs[(r + 2) % 3], sems[(r + 2) % 3])
                drain(sems[r % 3])
                extract_chunk(k, (r % 2) * CH, bufs[r % 3])

        nflushed = (NCH * CH) // OUTW
        for f in range(nflushed):
            @pl.when(k == (f + 1) * (OUTW // CH) - 1)
            def _flush(f=f):
                pltpu.sync_copy(
                    out_v, out_hbm.at[:, pl.ds(base + f * OUTW, OUTW)])
        return 0

    lax.fori_loop(0, NCH, body, 0)


def kernel(user_id, sex, user_table, sex_table):
    uid = user_id.astype(jnp.int32).reshape(NW, BPW)
    sx = sex.astype(jnp.int32).reshape(NW, BPW)
    utab_t = user_table.T
    stab = sex_table.reshape(64)
    out_t = _lookup_concat(uid, sx, utab_t, stab)
    return out_t.T
